# Initial kernel scaffold; baseline (speedup 1.0000x reference)
import jax
import jax.numpy as jnp
from jax.experimental import pallas as pl


def kernel(state):
    raise NotImplementedError("write your pallas kernel here")


# speedup vs baseline: 1.3629x; 1.3629x over previous
import jax
import jax.numpy as jnp
from jax.experimental import pallas as pl

WIDTH = 1000
FEATURE_DIM = 100000
COL_BLOCK = 2048


def _onehot_block(state_ref, out_ref):
    j = pl.program_id(0)
    idx = state_ref[:, 0] + WIDTH * state_ref[:, 1]
    cols = jax.lax.broadcasted_iota(jnp.int32, out_ref.shape, 1) + j * COL_BLOCK
    out_ref[...] = (cols == idx[:, None]).astype(jnp.float32)


def kernel(state):
    n = state.shape[0]
    grid = pl.cdiv(FEATURE_DIM, COL_BLOCK)
    return pl.pallas_call(
        _onehot_block,
        grid=(grid,),
        in_specs=[pl.BlockSpec((n, 2), lambda j: (0, 0))],
        out_specs=pl.BlockSpec((n, COL_BLOCK), lambda j: (0, j)),
        out_shape=jax.ShapeDtypeStruct((n, FEATURE_DIM), jnp.float32),
    )(state)


# speedup vs baseline: 1.3634x; 1.0004x over previous
import jax
import jax.numpy as jnp
from jax.experimental import pallas as pl

WIDTH = 1000
FEATURE_DIM = 100000
COL_BLOCK = 4096


def _onehot_block(state_ref, out_ref):
    j = pl.program_id(0)
    idx = state_ref[:, 0] + WIDTH * state_ref[:, 1]
    cols = jax.lax.broadcasted_iota(jnp.int32, out_ref.shape, 1) + j * COL_BLOCK
    out_ref[...] = (cols == idx[:, None]).astype(jnp.float32)


def kernel(state):
    n = state.shape[0]
    grid = pl.cdiv(FEATURE_DIM, COL_BLOCK)
    return pl.pallas_call(
        _onehot_block,
        grid=(grid,),
        in_specs=[pl.BlockSpec((n, 2), lambda j: (0, 0))],
        out_specs=pl.BlockSpec((n, COL_BLOCK), lambda j: (0, j)),
        out_shape=jax.ShapeDtypeStruct((n, FEATURE_DIM), jnp.float32),
    )(state)
